# Initial kernel scaffold; baseline (speedup 1.0000x reference)
import jax
import jax.numpy as jnp
from jax.experimental import pallas as pl


def kernel(tokens, token_embedding):
    raise NotImplementedError("write your pallas kernel here")


# speedup vs baseline: 2.9823x; 2.9823x over previous
import functools

import jax
import jax.numpy as jnp
from jax import lax
from jax.experimental import pallas as pl
from jax.experimental.pallas import tpu as pltpu
from jax.experimental.pallas import tpu_sc as plsc

VOCAB = 100000
EMBED_DIM = 128
TOKENS_SHAPE = (4096, 50)
B = TOKENS_SHAPE[0] * TOKENS_SHAPE[1]

NUM_CORES = 2
NUM_SUBCORES = 16
NW = NUM_CORES * NUM_SUBCORES
B_PER_W = B // NW
CHUNK = 128
N_CHUNKS = B_PER_W // CHUNK


def _emb_kernel(table_hbm, idx_hbm, out_hbm, idx_v, rows_v, gsem):
    wid = lax.axis_index("s") * NUM_CORES + lax.axis_index("c")
    base = wid * B_PER_W
    pltpu.sync_copy(idx_hbm.at[pl.ds(base, B_PER_W)], idx_v)

    def body(j, carry):
        pltpu.async_copy(
            table_hbm.at[idx_v.at[pl.ds(j * CHUNK, CHUNK)]],
            rows_v,
            gsem,
        ).wait()
        pltpu.sync_copy(rows_v, out_hbm.at[pl.ds(base + j * CHUNK, CHUNK)])
        return carry

    lax.fori_loop(0, N_CHUNKS, body, 0)


@functools.partial(jax.jit)
def _embedding_lookup(table, idx):
    mesh = plsc.VectorSubcoreMesh(core_axis_name="c", subcore_axis_name="s")
    return pl.kernel(
        _emb_kernel,
        out_type=jax.ShapeDtypeStruct((B, EMBED_DIM), jnp.float32),
        mesh=mesh,
        scratch_types=[
            pltpu.VMEM((B_PER_W,), jnp.int32),
            pltpu.VMEM((CHUNK, EMBED_DIM), jnp.float32),
            pltpu.SemaphoreType.DMA,
        ],
    )(table, idx)


def kernel(tokens, token_embedding):
    idx = tokens.reshape(B)
    out = _embedding_lookup(token_embedding, idx)
    return out.reshape(*TOKENS_SHAPE, EMBED_DIM)


# speedup vs baseline: 3.3470x; 1.1223x over previous
import functools

import jax
import jax.numpy as jnp
from jax import lax
from jax.experimental import pallas as pl
from jax.experimental.pallas import tpu as pltpu
from jax.experimental.pallas import tpu_sc as plsc

VOCAB = 100000
EMBED_DIM = 128
TOKENS_SHAPE = (4096, 50)
B = TOKENS_SHAPE[0] * TOKENS_SHAPE[1]

NUM_CORES = 2
NUM_SUBCORES = 16
NW = NUM_CORES * NUM_SUBCORES
B_PER_W = B // NW
CHUNK = 128
N_CHUNKS = B_PER_W // CHUNK


NBUF = 5
N_GROUPS = N_CHUNKS // NBUF


def _emb_kernel(table_hbm, idx_hbm, out_hbm, idx_v, rows_v, gsems, ssems):
    wid = lax.axis_index("s") * NUM_CORES + lax.axis_index("c")
    base = wid * B_PER_W
    pltpu.sync_copy(idx_hbm.at[pl.ds(base, B_PER_W)], idx_v)

    def gather(j, b):
        return pltpu.make_async_copy(
            table_hbm.at[idx_v.at[pl.ds(j * CHUNK, CHUNK)]],
            rows_v.at[b],
            gsems.at[b],
        )

    def scatter(j, b):
        return pltpu.make_async_copy(
            rows_v.at[b],
            out_hbm.at[pl.ds(base + j * CHUNK, CHUNK)],
            ssems.at[b],
        )

    for b in range(NBUF):
        gather(b, b).start()

    def group_body(gi, carry):
        j0 = gi * NBUF
        for b in range(NBUF):
            j = j0 + b
            gather(j, b).wait()
            scatter(j, b).start()
            scatter(j, b).wait()
            gather(j + NBUF, b).start()
        return carry

    lax.fori_loop(0, N_GROUPS - 1, group_body, 0)

    j0 = (N_GROUPS - 1) * NBUF
    for b in range(NBUF):
        gather(j0 + b, b).wait()
        scatter(j0 + b, b).start()
    for b in range(NBUF):
        scatter(j0 + b, b).wait()


@functools.partial(jax.jit)
def _embedding_lookup(table, idx):
    mesh = plsc.VectorSubcoreMesh(core_axis_name="c", subcore_axis_name="s")
    return pl.kernel(
        _emb_kernel,
        out_type=jax.ShapeDtypeStruct((B, EMBED_DIM), jnp.float32),
        mesh=mesh,
        scratch_types=[
            pltpu.VMEM((B_PER_W,), jnp.int32),
            pltpu.VMEM((NBUF, CHUNK, EMBED_DIM), jnp.float32),
            pltpu.SemaphoreType.DMA((NBUF,)),
            pltpu.SemaphoreType.DMA((NBUF,)),
        ],
    )(table, idx)


def kernel(tokens, token_embedding):
    idx = tokens.reshape(B)
    out = _embedding_lookup(token_embedding, idx)
    return out.reshape(*TOKENS_SHAPE, EMBED_DIM)
